# baseline (speedup 1.0000x reference)
import functools

import jax
import jax.numpy as jnp
from jax import lax
from jax.experimental import pallas as pl
from jax.experimental.pallas import tpu as pltpu
from jax.experimental.pallas import tpu_sc as plsc

VOCAB = 1000000
EMBED = 32

_info = plsc.get_sparse_core_info()
_NC, _NS = _info.num_cores, _info.num_subcores
_NW = _NC * _NS

_NB = 16384
_NT = 50
_BT = _NB // 128
_BT_W = _BT // _NW
_CT = EMBED // 8
_G = 5
_NG = _NT // _G


def _gather_kernel(idx_hbm, tab_hbm, out_hbm,
                   chunk_v, list0, list1, rows0, rows1, perm0, perm1,
                   gsem0, gsem1, wsem0, wsem1):
    wid = lax.axis_index("s") * _NC + lax.axis_index("c")
    iota = lax.broadcasted_iota(jnp.int32, (16,), 0)
    iota2 = iota + iota
    iota50 = iota * _NT

    def build(tg, list_x):
        for tt in range(_G):
            for mb in range(8):
                pos = mb * (16 * _NT) + iota50 + (tg * _G + tt)
                v = plsc.load_gather(chunk_v, [pos])
                list_x[pl.ds(tt * 128 + mb * 16, 16)] = v

    colvs = [iota * 0 + lo for lo in range(16)]
    rowvs = [lb * 16 + iota for lb in range(8)]
    hi16 = iota * 0 + 16

    def permute(rows_x, perm_x):
        def tt_body(tt, carry):
            rowts = [rowvs[lb] + tt * 128 for lb in range(8)]
            for lb in range(8):
                for hi in range(2):
                    vs = [plsc.load_gather(
                              rows_x,
                              [rowts[lb],
                               colvs[lo] + hi16 if hi else colvs[lo]])
                          for lo in range(16)]
                    for lo in range(16):
                        c = hi * 16 + lo
                        perm_x[tt, c // 8,
                               pl.ds((c % 8) * 128 + lb * 16, 16)] = vs[lo]
            return carry
        lax.fori_loop(0, _G, tt_body, 0)

    def bt_body(k, carry):
        bt = wid * _BT_W + k
        pltpu.sync_copy(idx_hbm.at[pl.ds(bt * (128 * _NT), 128 * _NT)],
                        chunk_v)
        build(0, list0)
        pltpu.async_copy(tab_hbm.at[list0], rows0, gsem0)

        def pair_body(i, carry2):
            g0 = 2 * i
            g1 = g0 + 1
            build(g1, list1)
            pltpu.async_copy(tab_hbm.at[list1], rows1, gsem1)
            pltpu.make_async_copy(tab_hbm.at[list0], rows0, gsem0).wait()

            @pl.when(i > 0)
            def _():
                pltpu.make_async_copy(perm0, out_hbm.at[pl.ds(g0 * _G, _G),
                                                        :, bt], wsem0).wait()
            permute(rows0, perm0)
            pltpu.async_copy(perm0, out_hbm.at[pl.ds(g0 * _G, _G), :, bt],
                             wsem0)

            @pl.when(i < _NG // 2 - 1)
            def _():
                build(g0 + 2, list0)
                pltpu.async_copy(tab_hbm.at[list0], rows0, gsem0)
            pltpu.make_async_copy(tab_hbm.at[list1], rows1, gsem1).wait()

            @pl.when(i > 0)
            def _():
                pltpu.make_async_copy(perm1, out_hbm.at[pl.ds(g1 * _G, _G),
                                                        :, bt], wsem1).wait()
            permute(rows1, perm1)
            pltpu.async_copy(perm1, out_hbm.at[pl.ds(g1 * _G, _G), :, bt],
                             wsem1)
            return carry2

        lax.fori_loop(0, _NG // 2, pair_body, 0)
        pltpu.make_async_copy(perm0, out_hbm.at[pl.ds(0, _G), :, bt],
                              wsem0).wait()
        pltpu.make_async_copy(perm1, out_hbm.at[pl.ds(0, _G), :, bt],
                              wsem1).wait()
        return carry

    lax.fori_loop(0, _BT_W, bt_body, 0)


@jax.jit
def _embed_lookup(idx_flat, tab16):
    mesh = plsc.VectorSubcoreMesh(core_axis_name="c", subcore_axis_name="s")
    kf = functools.partial(
        pl.kernel,
        mesh=mesh,
        out_type=jax.ShapeDtypeStruct((_NT, _CT, _BT, 1024), jnp.float32),
        scratch_types=[
            pltpu.VMEM((128 * _NT,), jnp.int32),
            pltpu.VMEM((_G * 128,), jnp.int32),
            pltpu.VMEM((_G * 128,), jnp.int32),
            pltpu.VMEM((_G * 128, EMBED), jnp.float32),
            pltpu.VMEM((_G * 128, EMBED), jnp.float32),
            pltpu.VMEM((_G, _CT, 1024), jnp.float32),
            pltpu.VMEM((_G, _CT, 1024), jnp.float32),
            pltpu.SemaphoreType.DMA,
            pltpu.SemaphoreType.DMA,
            pltpu.SemaphoreType.DMA,
            pltpu.SemaphoreType.DMA,
        ],
        compiler_params=pltpu.CompilerParams(
            use_tc_tiling_on_sc=False, needs_layout_passes=False),
    )(_gather_kernel)
    return kf(idx_flat, tab16)


def kernel(inputs, embeddings):
    idx_flat = inputs.astype(jnp.int32).reshape(-1)
    out5 = _embed_lookup(idx_flat, embeddings)
    return (out5.reshape(_NT, _CT, _BT, 8, 128)
            .transpose(2, 4, 0, 1, 3).reshape(_NB, _NT, EMBED))


# speedup vs baseline: 1.0003x; 1.0003x over previous
import functools

import jax
import jax.numpy as jnp
from jax import lax
from jax.experimental import pallas as pl
from jax.experimental.pallas import tpu as pltpu
from jax.experimental.pallas import tpu_sc as plsc

VOCAB = 1000000
EMBED = 32

_info = plsc.get_sparse_core_info()
_NC, _NS = _info.num_cores, _info.num_subcores
_NW = _NC * _NS

_NB = 16384
_NT = 50
_BT = _NB // 128
_BT_W = _BT // _NW
_CT = EMBED // 8
_G = 5
_NG = _NT // _G


def _gather_kernel(idx_hbm, tab_hbm, out_hbm,
                   chunk_v, list0, list1, rows0, rows1, perm0, perm1,
                   gsem0, gsem1, wsem0, wsem1):
    wid = lax.axis_index("s") * _NC + lax.axis_index("c")
    iota = lax.broadcasted_iota(jnp.int32, (16,), 0)
    iota2 = iota + iota
    iota50 = iota * _NT

    def build(tg, list_x):
        for tt in range(_G):
            for mb in range(8):
                pos = mb * (16 * _NT) + iota50 + (tg * _G + tt)
                v = plsc.load_gather(chunk_v, [pos])
                list_x[pl.ds(tt * 128 + mb * 16, 16)] = v

    colvs = [iota * 0 + lo for lo in range(16)]
    rowvs = [lb * 16 + iota for lb in range(8)]
    hi16 = iota * 0 + 16

    def permute(rows_x, perm_x):
        @plsc.parallel_loop(0, _G)
        def _(tt):
            rowts = [rowvs[lb] + tt * 128 for lb in range(8)]
            for lb in range(8):
                for hi in range(2):
                    vs = [plsc.load_gather(
                              rows_x,
                              [rowts[lb],
                               colvs[lo] + hi16 if hi else colvs[lo]])
                          for lo in range(16)]
                    for lo in range(16):
                        c = hi * 16 + lo
                        perm_x[tt, c // 8,
                               pl.ds((c % 8) * 128 + lb * 16, 16)] = vs[lo]

    def bt_body(k, carry):
        bt = wid * _BT_W + k
        pltpu.sync_copy(idx_hbm.at[pl.ds(bt * (128 * _NT), 128 * _NT)],
                        chunk_v)
        build(0, list0)
        pltpu.async_copy(tab_hbm.at[list0], rows0, gsem0)

        def pair_body(i, carry2):
            g0 = 2 * i
            g1 = g0 + 1
            build(g1, list1)
            pltpu.async_copy(tab_hbm.at[list1], rows1, gsem1)
            pltpu.make_async_copy(tab_hbm.at[list0], rows0, gsem0).wait()

            @pl.when(i > 0)
            def _():
                pltpu.make_async_copy(perm0, out_hbm.at[pl.ds(g0 * _G, _G),
                                                        :, bt], wsem0).wait()
            permute(rows0, perm0)
            pltpu.async_copy(perm0, out_hbm.at[pl.ds(g0 * _G, _G), :, bt],
                             wsem0)

            @pl.when(i < _NG // 2 - 1)
            def _():
                build(g0 + 2, list0)
                pltpu.async_copy(tab_hbm.at[list0], rows0, gsem0)
            pltpu.make_async_copy(tab_hbm.at[list1], rows1, gsem1).wait()

            @pl.when(i > 0)
            def _():
                pltpu.make_async_copy(perm1, out_hbm.at[pl.ds(g1 * _G, _G),
                                                        :, bt], wsem1).wait()
            permute(rows1, perm1)
            pltpu.async_copy(perm1, out_hbm.at[pl.ds(g1 * _G, _G), :, bt],
                             wsem1)
            return carry2

        lax.fori_loop(0, _NG // 2, pair_body, 0)
        pltpu.make_async_copy(perm0, out_hbm.at[pl.ds(0, _G), :, bt],
                              wsem0).wait()
        pltpu.make_async_copy(perm1, out_hbm.at[pl.ds(0, _G), :, bt],
                              wsem1).wait()
        return carry

    lax.fori_loop(0, _BT_W, bt_body, 0)


@jax.jit
def _embed_lookup(idx_flat, tab16):
    mesh = plsc.VectorSubcoreMesh(core_axis_name="c", subcore_axis_name="s")
    kf = functools.partial(
        pl.kernel,
        mesh=mesh,
        out_type=jax.ShapeDtypeStruct((_NT, _CT, _BT, 1024), jnp.float32),
        scratch_types=[
            pltpu.VMEM((128 * _NT,), jnp.int32),
            pltpu.VMEM((_G * 128,), jnp.int32),
            pltpu.VMEM((_G * 128,), jnp.int32),
            pltpu.VMEM((_G * 128, EMBED), jnp.float32),
            pltpu.VMEM((_G * 128, EMBED), jnp.float32),
            pltpu.VMEM((_G, _CT, 1024), jnp.float32),
            pltpu.VMEM((_G, _CT, 1024), jnp.float32),
            pltpu.SemaphoreType.DMA,
            pltpu.SemaphoreType.DMA,
            pltpu.SemaphoreType.DMA,
            pltpu.SemaphoreType.DMA,
        ],
        compiler_params=pltpu.CompilerParams(
            use_tc_tiling_on_sc=False, needs_layout_passes=False),
    )(_gather_kernel)
    return kf(idx_flat, tab16)


def kernel(inputs, embeddings):
    idx_flat = inputs.astype(jnp.int32).reshape(-1)
    out5 = _embed_lookup(idx_flat, embeddings)
    return (out5.reshape(_NT, _CT, _BT, 8, 128)
            .transpose(2, 4, 0, 1, 3).reshape(_NB, _NT, EMBED))
